# baseline (speedup 1.0000x reference)
import jax
import jax.numpy as jnp
import numpy as np
from jax.experimental import pallas as pl
from jax.experimental.pallas import tpu as pltpu

B, HORIZON, NUM_COV, HID = 8192, 24, 7, 512
INP = NUM_COV + 1
G4 = 4 * HID
KA = NUM_COV + 2
BB = 1024
NB = B // BB
NCHAIN = 4
CB = BB // NCHAIN
LG = 128
NJ = HID // LG

_PERM = np.concatenate(
    [np.concatenate([g * HID + np.arange(j * LG, (j + 1) * LG)
                     for g in range(4)]) for j in range(NJ)])


def _sigmoid(x):
    return 0.5 * jnp.tanh(0.5 * x) + 0.5


def _decoder_kernel(x_ref, h0_ref, c0_ref, y0_ref, wxa_ref, whh_ref,
                    fcw_ref, fcb_ref, out_ref):
    wxa = wxa_ref[...]
    whh = whh_ref[...]
    fcw = fcw_ref[...]
    fcb = fcb_ref[0, 0]
    ones_col = jnp.ones((CB, 1), jnp.bfloat16)

    hs = [h0_ref[q * CB:(q + 1) * CB, :].astype(jnp.bfloat16)
          for q in range(NCHAIN)]
    cs = [[c0_ref[q * CB:(q + 1) * CB, j * LG:(j + 1) * LG]
           for j in range(NJ)] for q in range(NCHAIN)]
    ys = [y0_ref[q * CB:(q + 1) * CB, :].astype(jnp.bfloat16)
          for q in range(NCHAIN)]

    for t in range(HORIZON):
        for q in range(NCHAIN):
            lo = q * CB
            x_aug = jnp.concatenate(
                [x_ref[lo:lo + CB, t * NUM_COV:(t + 1) * NUM_COV].astype(
                    jnp.bfloat16), ys[q], ones_col], axis=1)
            newh = []
            logit = fcb * jnp.ones((CB, 1), jnp.float32)
            for j in range(NJ):
                sl = slice(4 * LG * j, 4 * LG * (j + 1))
                gj = (
                    jnp.dot(hs[q], whh[:, sl],
                            preferred_element_type=jnp.float32)
                    + jnp.dot(x_aug, wxa[:, sl],
                              preferred_element_type=jnp.float32)
                )
                i = _sigmoid(gj[:, 0 * LG:1 * LG])
                f = _sigmoid(gj[:, 1 * LG:2 * LG])
                g = jnp.tanh(gj[:, 2 * LG:3 * LG])
                o = _sigmoid(gj[:, 3 * LG:4 * LG])
                cj = f * cs[q][j] + i * g
                cs[q][j] = cj
                hj = o * jnp.tanh(cj)
                newh.append(hj)
                logit = logit + jnp.sum(
                    hj * fcw[:, j * LG:(j + 1) * LG], axis=1, keepdims=True)
            hs[q] = jnp.concatenate(newh, axis=1).astype(jnp.bfloat16)
            ys[q] = _sigmoid(logit).astype(jnp.bfloat16)
            out_ref[lo:lo + CB, t:t + 1] = logit


def kernel(future_x, h_enc, c_enc, y0, W_ih, W_hh, b_ih, b_hh, fc_w, fc_b):
    perm = jnp.asarray(_PERM)
    wxa = jnp.concatenate(
        [W_ih.T, (b_ih + b_hh).reshape(1, G4)],
        axis=0)[:, perm].astype(jnp.bfloat16)
    whh = W_hh.T[:, perm].astype(jnp.bfloat16)
    fcb = fc_b.reshape(1, 1)

    out = pl.pallas_call(
        _decoder_kernel,
        grid=(NB,),
        in_specs=[
            pl.BlockSpec((BB, HORIZON * NUM_COV), lambda i: (i, 0)),
            pl.BlockSpec((BB, HID), lambda i: (i, 0)),
            pl.BlockSpec((BB, HID), lambda i: (i, 0)),
            pl.BlockSpec((BB, 1), lambda i: (i, 0)),
            pl.BlockSpec((KA, G4), lambda i: (0, 0)),
            pl.BlockSpec((HID, G4), lambda i: (0, 0)),
            pl.BlockSpec((1, HID), lambda i: (0, 0)),
            pl.BlockSpec((1, 1), lambda i: (0, 0)),
        ],
        out_specs=pl.BlockSpec((BB, HORIZON), lambda i: (i, 0)),
        out_shape=jax.ShapeDtypeStruct((B, HORIZON), jnp.float32),
        compiler_params=pltpu.CompilerParams(
            dimension_semantics=("parallel",),
            vmem_limit_bytes=56 * 1024 * 1024,
        ),
    )(future_x.reshape(B, HORIZON * NUM_COV), h_enc, c_enc, y0, wxa,
      whh, fc_w, fcb)
    return out[..., None]


# speedup vs baseline: 1.0089x; 1.0089x over previous
import jax
import jax.numpy as jnp
import numpy as np
from jax.experimental import pallas as pl
from jax.experimental.pallas import tpu as pltpu

B, HORIZON, NUM_COV, HID = 8192, 24, 7, 512
INP = NUM_COV + 1
G4 = 4 * HID
KA = NUM_COV + 2
BB = 1024
NB = B // BB
NCHAIN = 2
CB = BB // NCHAIN
LG = 128
NJ = HID // LG

_PERM = np.concatenate(
    [np.concatenate([g * HID + np.arange(j * LG, (j + 1) * LG)
                     for g in range(4)]) for j in range(NJ)])


def _sigmoid(x):
    return 0.5 * jnp.tanh(0.5 * x) + 0.5


def _decoder_kernel(x_ref, h0_ref, c0_ref, y0_ref, wxa_ref, whh_ref,
                    fcw_ref, fcb_ref, out_ref):
    wxa = wxa_ref[...]
    whh = whh_ref[...]
    fcw = fcw_ref[...]
    fcb = fcb_ref[0, 0]
    ones_col = jnp.ones((CB, 1), jnp.bfloat16)

    hs = [h0_ref[q * CB:(q + 1) * CB, :].astype(jnp.bfloat16)
          for q in range(NCHAIN)]
    cs = [[c0_ref[q * CB:(q + 1) * CB, j * LG:(j + 1) * LG]
           for j in range(NJ)] for q in range(NCHAIN)]
    ys = [y0_ref[q * CB:(q + 1) * CB, :].astype(jnp.bfloat16)
          for q in range(NCHAIN)]

    for t in range(HORIZON):
        for q in range(NCHAIN):
            lo = q * CB
            x_aug = jnp.concatenate(
                [x_ref[lo:lo + CB, t * NUM_COV:(t + 1) * NUM_COV].astype(
                    jnp.bfloat16), ys[q], ones_col], axis=1)
            newh = []
            logit = fcb * jnp.ones((CB, 1), jnp.float32)
            for j in range(NJ):
                sl = slice(4 * LG * j, 4 * LG * (j + 1))
                gj = (
                    jnp.dot(hs[q], whh[:, sl],
                            preferred_element_type=jnp.float32)
                    + jnp.dot(x_aug, wxa[:, sl],
                              preferred_element_type=jnp.float32)
                )
                i = _sigmoid(gj[:, 0 * LG:1 * LG])
                f = _sigmoid(gj[:, 1 * LG:2 * LG])
                g = jnp.tanh(gj[:, 2 * LG:3 * LG])
                o = _sigmoid(gj[:, 3 * LG:4 * LG])
                cj = f * cs[q][j] + i * g
                cs[q][j] = cj
                hj = o * jnp.tanh(cj)
                newh.append(hj)
                logit = logit + jnp.sum(
                    hj * fcw[:, j * LG:(j + 1) * LG], axis=1, keepdims=True)
            hs[q] = jnp.concatenate(newh, axis=1).astype(jnp.bfloat16)
            ys[q] = _sigmoid(logit).astype(jnp.bfloat16)
            out_ref[lo:lo + CB, t:t + 1] = logit


def kernel(future_x, h_enc, c_enc, y0, W_ih, W_hh, b_ih, b_hh, fc_w, fc_b):
    perm = jnp.asarray(_PERM)
    wxa = jnp.concatenate(
        [W_ih.T, (b_ih + b_hh).reshape(1, G4)],
        axis=0)[:, perm].astype(jnp.bfloat16)
    whh = W_hh.T[:, perm].astype(jnp.bfloat16)
    fcb = fc_b.reshape(1, 1)

    out = pl.pallas_call(
        _decoder_kernel,
        grid=(NB,),
        in_specs=[
            pl.BlockSpec((BB, HORIZON * NUM_COV), lambda i: (i, 0)),
            pl.BlockSpec((BB, HID), lambda i: (i, 0)),
            pl.BlockSpec((BB, HID), lambda i: (i, 0)),
            pl.BlockSpec((BB, 1), lambda i: (i, 0)),
            pl.BlockSpec((KA, G4), lambda i: (0, 0)),
            pl.BlockSpec((HID, G4), lambda i: (0, 0)),
            pl.BlockSpec((1, HID), lambda i: (0, 0)),
            pl.BlockSpec((1, 1), lambda i: (0, 0)),
        ],
        out_specs=pl.BlockSpec((BB, HORIZON), lambda i: (i, 0)),
        out_shape=jax.ShapeDtypeStruct((B, HORIZON), jnp.float32),
        compiler_params=pltpu.CompilerParams(
            dimension_semantics=("parallel",),
            vmem_limit_bytes=56 * 1024 * 1024,
        ),
    )(future_x.reshape(B, HORIZON * NUM_COV), h_enc, c_enc, y0, wxa,
      whh, fc_w, fcb)
    return out[..., None]
